# baseline (speedup 1.0000x reference)
import functools

import jax
import jax.numpy as jnp
from jax import lax
from jax.experimental import pallas as pl
from jax.experimental.pallas import tpu as pltpu
from jax.experimental.pallas import tpu_sc as plsc

_LANES = 16


def _make_sc_kernel(B, L, D, LP):
    info = plsc.get_sparse_core_info()
    NC, NS = info.num_cores, info.num_subcores
    NW = NC * NS
    RPW = B // NW
    ND = D // _LANES
    mesh = plsc.VectorSubcoreMesh(core_axis_name="c", subcore_axis_name="s")

    @functools.partial(
        pl.kernel,
        out_type=jax.ShapeDtypeStruct((B, D), jnp.float32),
        mesh=mesh,
        scratch_types=[
            pltpu.VMEM((L, D), jnp.float32),
            pltpu.VMEM((L, D), jnp.float32),
            pltpu.VMEM((LP,), jnp.float32),
            pltpu.VMEM((LP,), jnp.float32),
            pltpu.VMEM((RPW, D), jnp.float32),
            pltpu.SemaphoreType.DMA((2,)),
            pltpu.SemaphoreType.DMA((2,)),
        ],
    )
    def k(x_hbm, m_hbm, out_hbm, xb0, xb1, mb0, mb1, obuf, xsem, msem):
        wid = lax.axis_index("s") * NC + lax.axis_index("c")
        base = wid * RPW
        xbufs = (xb0, xb1)
        mbufs = (mb0, mb1)

        def start(r, slot):
            pltpu.async_copy(x_hbm.at[base + r], xbufs[slot], xsem.at[slot])
            pltpu.async_copy(m_hbm.at[base + r], mbufs[slot], msem.at[slot])

        def wait(slot):
            pltpu.make_async_copy(x_hbm.at[base], xbufs[slot], xsem.at[slot]).wait()
            pltpu.make_async_copy(m_hbm.at[base], mbufs[slot], msem.at[slot]).wait()

        def compute(r, slot):
            xbuf = xbufs[slot]
            mbuf = mbufs[slot]
            zero = jnp.zeros((_LANES,), jnp.float32)
            nfull = L // _LANES
            ntail = L - nfull * _LANES

            def chunk_body(c, carry, njs=_LANES):
                cnt = carry[0]
                accs = list(carry[1:])
                chunk = mbuf[pl.ds(c * _LANES, _LANES)]
                for j in range(njs):
                    ms = chunk[j]
                    l = c * _LANES + j
                    cnt = cnt + ms
                    for db in range(ND):
                        xv = xbuf[l, pl.ds(db * _LANES, _LANES)]
                        accs[db] = accs[db] + xv * ms
                return (cnt, *accs)

            carry = (jnp.float32(0),) + (zero,) * ND
            carry = lax.fori_loop(0, nfull, chunk_body, carry)
            if ntail:
                carry = chunk_body(nfull, carry, njs=ntail)
            cnt = carry[0]
            for db in range(ND):
                obuf[r, pl.ds(db * _LANES, _LANES)] = carry[db + 1] / cnt

        start(0, 0)

        def outer(i, _):
            r0 = i * 2
            start(r0 + 1, 1)
            wait(0)
            compute(r0, 0)

            @pl.when(r0 + 2 < RPW)
            def _():
                start(r0 + 2, 0)

            wait(1)
            compute(r0 + 1, 1)
            return 0

        lax.fori_loop(0, RPW // 2, outer, 0)
        pltpu.sync_copy(obuf, out_hbm.at[pl.ds(base, RPW)])

    return k


def kernel(inputs, mask):
    B, L, D = inputs.shape
    LP = 208
    mf = jnp.pad(mask.astype(inputs.dtype), ((0, 0), (0, LP - L)))
    return _make_sc_kernel(B, L, D, LP)(inputs, mf)


# speedup vs baseline: 1.3454x; 1.3454x over previous
import functools

import jax
import jax.numpy as jnp
from jax import lax
from jax.experimental import pallas as pl
from jax.experimental.pallas import tpu as pltpu
from jax.experimental.pallas import tpu_sc as plsc

_LANES = 16
_BB = 128
_TC_ROWS = 2560


def _tc_body(x_ref, m_ref, o_ref):
    x = x_ref[...]
    m = m_ref[...].astype(x.dtype)
    s = jnp.sum(x * m[:, :, None], axis=1)
    c = jnp.sum(m, axis=1, keepdims=True)
    o_ref[...] = s / c


def _tc_call(inputs, mask_u8, nrows):
    B, L, D = inputs.shape
    return pl.pallas_call(
        _tc_body,
        grid=(nrows // _BB,),
        in_specs=[
            pl.BlockSpec((_BB, L, D), lambda i: (i, 0, 0)),
            pl.BlockSpec((_BB, L), lambda i: (i, 0)),
        ],
        out_specs=pl.BlockSpec((_BB, D), lambda i: (i, 0)),
        out_shape=jax.ShapeDtypeStruct((nrows, D), inputs.dtype),
    )(inputs, mask_u8)


def _make_sc_kernel(B, L, D, LP, row0, nrows):
    info = plsc.get_sparse_core_info()
    NC, NS = info.num_cores, info.num_subcores
    NW = NC * NS
    RPW = nrows // NW
    ND = D // _LANES
    mesh = plsc.VectorSubcoreMesh(core_axis_name="c", subcore_axis_name="s")

    @functools.partial(
        pl.kernel,
        out_type=jax.ShapeDtypeStruct((nrows, D), jnp.float32),
        mesh=mesh,
        scratch_types=[
            pltpu.VMEM((L, D), jnp.float32),
            pltpu.VMEM((L, D), jnp.float32),
            pltpu.VMEM((LP,), jnp.float32),
            pltpu.VMEM((LP,), jnp.float32),
            pltpu.VMEM((RPW, D), jnp.float32),
            pltpu.SemaphoreType.DMA((2,)),
            pltpu.SemaphoreType.DMA((2,)),
        ],
    )
    def k(x_hbm, m_hbm, out_hbm, xb0, xb1, mb0, mb1, obuf, xsem, msem):
        wid = lax.axis_index("s") * NC + lax.axis_index("c")
        obase = wid * RPW
        base = row0 + obase
        xbufs = (xb0, xb1)
        mbufs = (mb0, mb1)

        def start(r, slot):
            pltpu.async_copy(x_hbm.at[base + r], xbufs[slot], xsem.at[slot])
            pltpu.async_copy(m_hbm.at[base + r], mbufs[slot], msem.at[slot])

        def wait(slot):
            pltpu.make_async_copy(x_hbm.at[base], xbufs[slot], xsem.at[slot]).wait()
            pltpu.make_async_copy(m_hbm.at[base], mbufs[slot], msem.at[slot]).wait()

        def compute(r, slot):
            xbuf = xbufs[slot]
            mbuf = mbufs[slot]
            zero = jnp.zeros((_LANES,), jnp.float32)
            nfull = L // _LANES
            ntail = L - nfull * _LANES

            def chunk_body(c, carry, njs=_LANES):
                cnt = carry[0]
                accs = list(carry[1:])
                chunk = mbuf[pl.ds(c * _LANES, _LANES)]
                for j in range(njs):
                    ms = chunk[j]
                    l = c * _LANES + j
                    cnt = cnt + ms
                    for db in range(ND):
                        xv = xbuf[l, pl.ds(db * _LANES, _LANES)]
                        accs[db] = accs[db] + xv * ms
                return (cnt, *accs)

            carry = (jnp.float32(0),) + (zero,) * ND
            carry = lax.fori_loop(0, nfull, chunk_body, carry)
            if ntail:
                carry = chunk_body(nfull, carry, njs=ntail)
            cnt = carry[0]
            for db in range(ND):
                obuf[r, pl.ds(db * _LANES, _LANES)] = carry[db + 1] / cnt

        start(0, 0)

        def outer(i, _):
            r0 = i * 2
            start(r0 + 1, 1)
            wait(0)
            compute(r0, 0)

            @pl.when(r0 + 2 < RPW)
            def _():
                start(r0 + 2, 0)

            wait(1)
            compute(r0 + 1, 1)
            return 0

        lax.fori_loop(0, RPW // 2, outer, 0)
        pltpu.sync_copy(obuf, out_hbm.at[pl.ds(obase, RPW)])

    return k


def kernel(inputs, mask):
    B, L, D = inputs.shape
    LP = 208
    mf = jnp.pad(mask.astype(inputs.dtype), ((0, 0), (0, LP - L)))
    out_tc = _tc_call(inputs, mask.view(jnp.uint8), _TC_ROWS)
    out_sc = _make_sc_kernel(B, L, D, LP, _TC_ROWS, B - _TC_ROWS)(inputs, mf)
    return jnp.concatenate([out_tc, out_sc], axis=0)
